# baseline (speedup 1.0000x reference)
import functools

import jax
import jax.numpy as jnp
from jax import lax
from jax.experimental import pallas as pl
from jax.experimental.pallas import tpu as pltpu
from jax.experimental.pallas import tpu_sc as plsc

N = 10000
D = 128
E = 320000
EPS = 1e-5

NW = 32
R = 313
NPAD = NW * R
BLK = 2000
NBLK = E // BLK
C = 128
NSLOT = 4
NEG = float("-inf")

_i32 = jnp.int32
_f32 = jnp.float32


def _segmax_body(z, srca, dsta, m, acc, sbuf, dbuf, ssrc, sdst, rows,
                 semE0, semE1, semE2, semE3, semG0, semG1, semX, wpr):
  wid = lax.axis_index("s") * 2 + lax.axis_index("c")
  base = wid * R
  semE = (semE0, semE1, semE2, semE3)

  neg16 = jnp.full((16,), NEG, dtype=_f32)
  def _ini(i, _):
    acc[pl.ds(i * 16, 16)] = neg16
    return 0
  lax.fori_loop(0, (R + 1) * D // 16, _ini, 0)

  z16 = jnp.zeros((16,), dtype=_i32)
  r16 = jnp.full((16,), R, dtype=_i32)
  for i in range(NSLOT * C // 16):
    ssrc[pl.ds(i * 16, 16)] = z16
  for i in range((NSLOT * C + 16) // 16):
    sdst[pl.ds(i * 16, 16)] = r16
  wpr[0] = 0

  def _startE(b, s):
    pltpu.async_copy(srca.at[pl.ds(b * BLK, BLK)],
                     sbuf.at[pl.ds(s * BLK, BLK)], semE[s])
    pltpu.async_copy(dsta.at[pl.ds(b * BLK, BLK)],
                     dbuf.at[pl.ds(s * BLK, BLK)], semE[s])

  def _waitE(s):
    pltpu.make_async_copy(srca.at[pl.ds(0, BLK)],
                          sbuf.at[pl.ds(s * BLK, BLK)], semE[s]).wait()
    pltpu.make_async_copy(dsta.at[pl.ds(0, BLK)],
                          dbuf.at[pl.ds(s * BLK, BLK)], semE[s]).wait()

  def _fireG(s, sem):
    pltpu.async_copy(z.at[ssrc.at[pl.ds(s * C, C)]],
                     rows.at[pl.ds(s * C, C)], sem)

  def _waitG(sem):
    pltpu.make_async_copy(z.at[ssrc.at[pl.ds(0, C)]],
                          rows.at[pl.ds(0, C)], sem).wait()

  def _accum(s):
    so = s * C
    def _edge(j, _):
      d = sdst[pl.ds(so + j, 16)][0]
      ab = d * D
      for k in range(D // 16):
        a = acc[pl.ds(ab + k * 16, 16)]
        r = rows[so + j, pl.ds(k * 16, 16)]
        acc[pl.ds(ab + k * 16, 16)] = jnp.maximum(a, r)
      return 0
    lax.fori_loop(0, C, _edge, 0)

  def _process(es, gs):
    def _vec(v, _):
      off = es * BLK + v * 16
      d16 = dbuf[pl.ds(off, 16)]
      msk = (d16 >= base) & (d16 < base + R)
      n = plsc.all_reduce_population_count(msk)[0]
      @pl.when(n > 0)
      def _():
        wp = wpr[0]
        s16 = sbuf[pl.ds(off, 16)]
        mi = jnp.where(msk, 1, 0).astype(_i32)
        cum = plsc.cumsum(mi)
        dest = cum - mi + (gs * C + wp)
        plsc.store_scatter(ssrc, [dest], s16, mask=msk)
        plsc.store_scatter(sdst, [dest], d16 - base, mask=msk)
        wp2 = wp + n
        @pl.when(wp2 > C - 16)
        def _():
          _fireG(gs, semX)
          _waitG(semX)
          _accum(gs)
        wpr[0] = jnp.where(wp2 > C - 16, 0, wp2)
      return 0
    lax.fori_loop(0, BLK // 16, _vec, 0)

  for s in range(NSLOT):
    _startE(s, s)

  def _blk4(bb, _):
    for u in range(NSLOT):
      b = 4 * bb + u
      _waitE(u)
      _process(u, u)
      @pl.when(b >= 2)
      def _():
        _waitG(semG0 if u % 2 == 0 else semG1)
        _accum((u + 2) % NSLOT)
      _fireG(u, semG0 if u % 2 == 0 else semG1)
      wpr[0] = 0
      @pl.when(b + NSLOT < NBLK)
      def _():
        _startE(b + NSLOT, u)
    return 0

  lax.fori_loop(0, NBLK // NSLOT, _blk4, 0)

  _waitG(semG0)
  _accum(2)
  _waitG(semG1)
  _accum(3)

  pltpu.sync_copy(acc.at[pl.ds(0, R * D)], m.at[pl.ds(base * D, R * D)])


@functools.partial(
    pl.kernel,
    out_type=jax.ShapeDtypeStruct((NPAD * D,), _f32),
    mesh=plsc.VectorSubcoreMesh(core_axis_name="c", subcore_axis_name="s"),
    compiler_params=pltpu.CompilerParams(needs_layout_passes=False),
    scratch_types=[
        pltpu.VMEM(((R + 1) * D,), _f32),
        pltpu.VMEM((NSLOT * BLK,), _i32),
        pltpu.VMEM((NSLOT * BLK,), _i32),
        pltpu.VMEM((NSLOT * C,), _i32),
        pltpu.VMEM((NSLOT * C + 16,), _i32),
        pltpu.VMEM((NSLOT * C, D), _f32),
        pltpu.SemaphoreType.DMA,
        pltpu.SemaphoreType.DMA,
        pltpu.SemaphoreType.DMA,
        pltpu.SemaphoreType.DMA,
        pltpu.SemaphoreType.DMA,
        pltpu.SemaphoreType.DMA,
        pltpu.SemaphoreType.DMA,
        pltpu.SMEM((1,), _i32),
    ],
)
def _segmax(z, srca, dsta, m, acc, sbuf, dbuf, ssrc, sdst, rows,
            semE0, semE1, semE2, semE3, semG0, semG1, semX, wpr):
  _segmax_body(z, srca, dsta, m, acc, sbuf, dbuf, ssrc, sdst, rows,
               semE0, semE1, semE2, semE3, semG0, semG1, semX, wpr)


def _bn(h, g, be):
  mu = jnp.mean(h, axis=0)
  va = jnp.var(h, axis=0)
  return (h - mu) / jnp.sqrt(va + EPS) * g + be


def _prep_body(x_r, pos2_r, w1x_r, w1p_r, wl_r, bl_r, gl_r, bel_r,
               z1_r, p1_r, skip_r):
  x = x_r[...]
  pos2 = pos2_r[...]
  p1 = jnp.dot(pos2, w1p_r[...], preferred_element_type=_f32)
  z1_r[...] = jnp.dot(x, w1x_r[...], preferred_element_type=_f32) + p1
  p1_r[...] = p1
  xl = jnp.dot(x, wl_r[...], preferred_element_type=_f32) + bl_r[...]
  skip_r[...] = _bn(xl, gl_r[...], bel_r[...])


def _mid_body(m1_r, p1_r, b1_r, g1_r, be1_r, pos2_r, w2x_r, w2p_r,
              z2_r, p2_r):
  agg = m1_r[...]
  c1 = jnp.where(jnp.isneginf(agg), 0.0, agg - p1_r[...] + b1_r[...])
  h = jax.nn.relu(_bn(c1, g1_r[...], be1_r[...]))
  p2 = jnp.dot(pos2_r[...], w2p_r[...], preferred_element_type=_f32)
  z2_r[...] = jnp.dot(h, w2x_r[...], preferred_element_type=_f32) + p2
  p2_r[...] = p2


def _fin_body(m2_r, p2_r, b2_r, g2_r, be2_r, skip_r, out_r):
  agg = m2_r[...]
  c2 = jnp.where(jnp.isneginf(agg), 0.0, agg - p2_r[...] + b2_r[...])
  out_r[...] = jax.nn.relu(_bn(c2, g2_r[...], be2_r[...]) + skip_r[...])


def _tc_call(body, n_out):
  return pl.pallas_call(
      body,
      out_shape=tuple(jax.ShapeDtypeStruct((N, D), _f32)
                      for _ in range(n_out)),
  )


def kernel(x, pos, edge_index, W1, b1, g1, be1, W2, b2, g2, be2,
           Wl, bl, gl, bel):
  pos2 = pos[:, :2]
  src = edge_index[0].astype(_i32)
  dst = edge_index[1].astype(_i32)
  b1_, g1_, be1_ = b1.reshape(1, D), g1.reshape(1, D), be1.reshape(1, D)
  b2_, g2_, be2_ = b2.reshape(1, D), g2.reshape(1, D), be2.reshape(1, D)
  bl_, gl_, bel_ = bl.reshape(1, D), gl.reshape(1, D), bel.reshape(1, D)

  z1, p1, skip = _tc_call(_prep_body, 3)(
      x, pos2, W1[:D], W1[D:], Wl, bl_, gl_, bel_)

  m1 = _segmax(z1, src, dst).reshape(NPAD, D)[:N]

  z2, p2 = _tc_call(_mid_body, 2)(
      m1, p1, b1_, g1_, be1_, pos2, W2[:D], W2[D:])

  m2 = _segmax(z2, src, dst).reshape(NPAD, D)[:N]

  (out,) = _tc_call(_fin_body, 1)(m2, p2, b2_, g2_, be2_, skip)
  return out


# speedup vs baseline: 1.0041x; 1.0041x over previous
import functools

import jax
import jax.numpy as jnp
from jax import lax
from jax.experimental import pallas as pl
from jax.experimental.pallas import tpu as pltpu
from jax.experimental.pallas import tpu_sc as plsc

N = 10000
D = 128
E = 320000
EPS = 1e-5

NW = 32
R = 313
NPAD = NW * R
BLK = 2000
NBLK = E // BLK
C = 128
NSLOT = 4
NEG = float("-inf")

_i32 = jnp.int32
_f32 = jnp.float32


def _segmax_body(z, srca, dsta, m, acc, sbuf, dbuf, ssrc, sdst, rows,
                 semE0, semE1, semE2, semE3, semG0, semG1, semX):
  wid = lax.axis_index("s") * 2 + lax.axis_index("c")
  base = wid * R
  semE = (semE0, semE1, semE2, semE3)

  neg16 = jnp.full((16,), NEG, dtype=_f32)
  def _ini(i, _):
    acc[pl.ds(i * 16, 16)] = neg16
    return 0
  lax.fori_loop(0, (R + 1) * D // 16, _ini, 0)

  z16 = jnp.zeros((16,), dtype=_i32)
  r16 = jnp.full((16,), R, dtype=_i32)
  for i in range(NSLOT * C // 16):
    ssrc[pl.ds(i * 16, 16)] = z16
  for i in range((NSLOT * C + 16) // 16):
    sdst[pl.ds(i * 16, 16)] = r16
  def _startE(b, s):
    pltpu.async_copy(srca.at[pl.ds(b * BLK, BLK)],
                     sbuf.at[pl.ds(s * BLK, BLK)], semE[s])
    pltpu.async_copy(dsta.at[pl.ds(b * BLK, BLK)],
                     dbuf.at[pl.ds(s * BLK, BLK)], semE[s])

  def _waitE(s):
    pltpu.make_async_copy(srca.at[pl.ds(0, BLK)],
                          sbuf.at[pl.ds(s * BLK, BLK)], semE[s]).wait()
    pltpu.make_async_copy(dsta.at[pl.ds(0, BLK)],
                          dbuf.at[pl.ds(s * BLK, BLK)], semE[s]).wait()

  def _fireG(s, sem):
    pltpu.async_copy(z.at[ssrc.at[pl.ds(s * C, C)]],
                     rows.at[pl.ds(s * C, C)], sem)

  def _waitG(sem):
    pltpu.make_async_copy(z.at[ssrc.at[pl.ds(0, C)]],
                          rows.at[pl.ds(0, C)], sem).wait()

  def _accum(s):
    so = s * C
    def _grp(gg, _):
      d16 = sdst[pl.ds(so + gg * 16, 16)]
      for jj in range(16):
        ab = d16[jj] * D
        for k in range(D // 16):
          a = acc[pl.ds(ab + k * 16, 16)]
          r = rows[so + gg * 16 + jj, pl.ds(k * 16, 16)]
          acc[pl.ds(ab + k * 16, 16)] = jnp.maximum(a, r)
      return 0
    lax.fori_loop(0, C // 16, _grp, 0)

  def _process(es, gs, wp0):
    def _vec(v, wp):
      off = es * BLK + v * 16
      d16 = dbuf[pl.ds(off, 16)]
      s16 = sbuf[pl.ds(off, 16)]
      msk = (d16 >= base) & (d16 < base + R)
      mi = jnp.where(msk, 1, 0).astype(_i32)
      cum = plsc.cumsum(mi)
      dest = cum - mi + (gs * C + wp)
      plsc.store_scatter(ssrc, [dest], s16, mask=msk)
      plsc.store_scatter(sdst, [dest], d16 - base, mask=msk)
      wp2 = wp + jnp.sum(mi)
      cond = wp2 > C - 16
      @pl.when(cond)
      def _():
        _fireG(gs, semX)
        _waitG(semX)
        _accum(gs)
      return jnp.where(cond, 0, wp2)
    return lax.fori_loop(0, BLK // 16, _vec, wp0)

  for s in range(NSLOT):
    _startE(s, s)

  def _blk4(bb, _):
    for u in range(NSLOT):
      b = 4 * bb + u
      _waitE(u)
      _process(u, u, 0)
      @pl.when(b >= 2)
      def _():
        _waitG(semG0 if u % 2 == 0 else semG1)
        _accum((u + 2) % NSLOT)
      _fireG(u, semG0 if u % 2 == 0 else semG1)
      @pl.when(b + NSLOT < NBLK)
      def _():
        _startE(b + NSLOT, u)
    return 0

  lax.fori_loop(0, NBLK // NSLOT, _blk4, 0)

  _waitG(semG0)
  _accum(2)
  _waitG(semG1)
  _accum(3)

  pltpu.sync_copy(acc.at[pl.ds(0, R * D)], m.at[pl.ds(base * D, R * D)])


@functools.partial(
    pl.kernel,
    out_type=jax.ShapeDtypeStruct((NPAD * D,), _f32),
    mesh=plsc.VectorSubcoreMesh(core_axis_name="c", subcore_axis_name="s"),
    compiler_params=pltpu.CompilerParams(needs_layout_passes=False),
    scratch_types=[
        pltpu.VMEM(((R + 1) * D,), _f32),
        pltpu.VMEM((NSLOT * BLK,), _i32),
        pltpu.VMEM((NSLOT * BLK,), _i32),
        pltpu.VMEM((NSLOT * C,), _i32),
        pltpu.VMEM((NSLOT * C + 16,), _i32),
        pltpu.VMEM((NSLOT * C, D), _f32),
        pltpu.SemaphoreType.DMA,
        pltpu.SemaphoreType.DMA,
        pltpu.SemaphoreType.DMA,
        pltpu.SemaphoreType.DMA,
        pltpu.SemaphoreType.DMA,
        pltpu.SemaphoreType.DMA,
        pltpu.SemaphoreType.DMA,
    ],
)
def _segmax(z, srca, dsta, m, acc, sbuf, dbuf, ssrc, sdst, rows,
            semE0, semE1, semE2, semE3, semG0, semG1, semX):
  _segmax_body(z, srca, dsta, m, acc, sbuf, dbuf, ssrc, sdst, rows,
               semE0, semE1, semE2, semE3, semG0, semG1, semX)


def _bn(h, g, be):
  mu = jnp.mean(h, axis=0)
  va = jnp.var(h, axis=0)
  return (h - mu) / jnp.sqrt(va + EPS) * g + be


def _prep_body(x_r, pos2_r, w1x_r, w1p_r, wl_r, bl_r, gl_r, bel_r,
               z1_r, p1_r, skip_r):
  x = x_r[...]
  pos2 = pos2_r[...]
  p1 = jnp.dot(pos2, w1p_r[...], preferred_element_type=_f32)
  z1_r[...] = jnp.dot(x, w1x_r[...], preferred_element_type=_f32) + p1
  p1_r[...] = p1
  xl = jnp.dot(x, wl_r[...], preferred_element_type=_f32) + bl_r[...]
  skip_r[...] = _bn(xl, gl_r[...], bel_r[...])


def _mid_body(m1_r, p1_r, b1_r, g1_r, be1_r, pos2_r, w2x_r, w2p_r,
              z2_r, p2_r):
  agg = m1_r[...]
  c1 = jnp.where(jnp.isneginf(agg), 0.0, agg - p1_r[...] + b1_r[...])
  h = jax.nn.relu(_bn(c1, g1_r[...], be1_r[...]))
  p2 = jnp.dot(pos2_r[...], w2p_r[...], preferred_element_type=_f32)
  z2_r[...] = jnp.dot(h, w2x_r[...], preferred_element_type=_f32) + p2
  p2_r[...] = p2


def _fin_body(m2_r, p2_r, b2_r, g2_r, be2_r, skip_r, out_r):
  agg = m2_r[...]
  c2 = jnp.where(jnp.isneginf(agg), 0.0, agg - p2_r[...] + b2_r[...])
  out_r[...] = jax.nn.relu(_bn(c2, g2_r[...], be2_r[...]) + skip_r[...])


def _tc_call(body, n_out):
  return pl.pallas_call(
      body,
      out_shape=tuple(jax.ShapeDtypeStruct((N, D), _f32)
                      for _ in range(n_out)),
  )


def kernel(x, pos, edge_index, W1, b1, g1, be1, W2, b2, g2, be2,
           Wl, bl, gl, bel):
  pos2 = pos[:, :2]
  src = edge_index[0].astype(_i32)
  dst = edge_index[1].astype(_i32)
  b1_, g1_, be1_ = b1.reshape(1, D), g1.reshape(1, D), be1.reshape(1, D)
  b2_, g2_, be2_ = b2.reshape(1, D), g2.reshape(1, D), be2.reshape(1, D)
  bl_, gl_, bel_ = bl.reshape(1, D), gl.reshape(1, D), bel.reshape(1, D)

  z1, p1, skip = _tc_call(_prep_body, 3)(
      x, pos2, W1[:D], W1[D:], Wl, bl_, gl_, bel_)

  m1 = _segmax(z1, src, dst).reshape(NPAD, D)[:N]

  z2, p2 = _tc_call(_mid_body, 2)(
      m1, p1, b1_, g1_, be1_, pos2, W2[:D], W2[D:])

  m2 = _segmax(z2, src, dst).reshape(NPAD, D)[:N]

  (out,) = _tc_call(_fin_body, 1)(m2, p2, b2_, g2_, be2_, skip)
  return out


# speedup vs baseline: 11.0410x; 10.9960x over previous
import functools

import jax
import jax.numpy as jnp
from jax import lax
from jax.experimental import pallas as pl
from jax.experimental.pallas import tpu as pltpu
from jax.experimental.pallas import tpu_sc as plsc

N = 10000
D = 128
E = 320000
EPS = 1e-5

NW = 32
R = 313
NPAD = NW * R
BLK = 4000
NBLK = E // BLK
C = 128
K = 4
CC = K * C
NEG = float("-inf")

_i32 = jnp.int32
_f32 = jnp.float32


def _segmax_body(z, srca, dsta, m, acc, sbuf, dbuf, ssrc, sdst, rows,
                 semA, semB, semg):
  wid = lax.axis_index("s") * 2 + lax.axis_index("c")
  base = wid * R

  neg16 = jnp.full((16,), NEG, dtype=_f32)
  def _ini(i, _):
    acc[pl.ds(i * 16, 16)] = neg16
    return 0
  lax.fori_loop(0, (R + 1) * D // 16, _ini, 0)

  z16 = jnp.zeros((16,), dtype=_i32)
  r16 = jnp.full((16,), R, dtype=_i32)
  for i in range(CC // 16):
    ssrc[pl.ds(i * 16, 16)] = z16
    sdst[pl.ds(i * 16, 16)] = r16

  def _start(b, soff, sem):
    pltpu.async_copy(srca.at[pl.ds(b * BLK, BLK)],
                     sbuf.at[pl.ds(soff, BLK)], sem)
    pltpu.async_copy(dsta.at[pl.ds(b * BLK, BLK)],
                     dbuf.at[pl.ds(soff, BLK)], sem)

  def _wait(soff, sem):
    pltpu.make_async_copy(srca.at[pl.ds(0, BLK)],
                          sbuf.at[pl.ds(soff, BLK)], sem).wait()
    pltpu.make_async_copy(dsta.at[pl.ds(0, BLK)],
                          dbuf.at[pl.ds(soff, BLK)], sem).wait()

  def _drain():
    for r in range(K):
      pltpu.async_copy(z.at[ssrc.at[pl.ds(r * C, C)]],
                       rows.at[pl.ds(r * C, C)], semg)
    def _grp(gg, _):
      d16 = sdst[pl.ds(gg * 16, 16)]
      for jj in range(16):
        ab = d16[jj] * D
        for k in range(D // 16):
          a = acc[pl.ds(ab + k * 16, 16)]
          r = rows[gg * 16 + jj, pl.ds(k * 16, 16)]
          acc[pl.ds(ab + k * 16, 16)] = jnp.maximum(a, r)
      return 0
    for r in range(K):
      pltpu.make_async_copy(z.at[ssrc.at[pl.ds(0, C)]],
                            rows.at[pl.ds(0, C)], semg).wait()
      lax.fori_loop(r * (C // 16), (r + 1) * (C // 16), _grp, 0)

  def _process(soff, wp0):
    def _vec(v, wp):
      off = soff + v * 16
      d16 = dbuf[pl.ds(off, 16)]
      s16 = sbuf[pl.ds(off, 16)]
      msk = (d16 >= base) & (d16 < base + R)
      mi = jnp.where(msk, 1, 0).astype(_i32)
      cum = plsc.cumsum(mi)
      dest = cum - mi + wp
      plsc.store_scatter(ssrc, [dest], s16, mask=msk)
      plsc.store_scatter(sdst, [dest], d16 - base, mask=msk)
      wp2 = wp + plsc.all_reduce_population_count(msk)[0]
      cond = wp2 > CC - 16
      @pl.when(cond)
      def _():
        _drain()
      return jnp.where(cond, 0, wp2)
    return lax.fori_loop(0, BLK // 16, _vec, wp0)

  _start(0, 0, semA)

  def _blk2(bb, wp):
    b1 = 2 * bb + 1
    _wait(0, semA)
    _start(b1, BLK, semB)
    wp = _process(0, wp)
    _wait(BLK, semB)
    @pl.when(b1 + 1 < NBLK)
    def _():
      _start(b1 + 1, 0, semA)
    wp = _process(BLK, wp)
    return wp

  wp = lax.fori_loop(0, NBLK // 2, _blk2, 0)

  @pl.when(wp > 0)
  def _():
    _drain()

  pltpu.sync_copy(acc.at[pl.ds(0, R * D)], m.at[pl.ds(base * D, R * D)])


@functools.partial(
    pl.kernel,
    out_type=jax.ShapeDtypeStruct((NPAD * D,), _f32),
    mesh=plsc.VectorSubcoreMesh(core_axis_name="c", subcore_axis_name="s"),
    compiler_params=pltpu.CompilerParams(needs_layout_passes=False),
    scratch_types=[
        pltpu.VMEM(((R + 1) * D,), _f32),
        pltpu.VMEM((2 * BLK,), _i32),
        pltpu.VMEM((2 * BLK,), _i32),
        pltpu.VMEM((CC,), _i32),
        pltpu.VMEM((CC,), _i32),
        pltpu.VMEM((CC, D), _f32),
        pltpu.SemaphoreType.DMA,
        pltpu.SemaphoreType.DMA,
        pltpu.SemaphoreType.DMA,
    ],
)
def _segmax(z, srca, dsta, m, acc, sbuf, dbuf, ssrc, sdst, rows,
            semA, semB, semg):
  _segmax_body(z, srca, dsta, m, acc, sbuf, dbuf, ssrc, sdst, rows,
               semA, semB, semg)


def _bn(h, g, be):
  mu = jnp.mean(h, axis=0)
  va = jnp.var(h, axis=0)
  return (h - mu) / jnp.sqrt(va + EPS) * g + be


def _prep_body(x_r, pos2_r, w1x_r, w1p_r, wl_r, bl_r, gl_r, bel_r,
               z1_r, p1_r, skip_r):
  x = x_r[...]
  pos2 = pos2_r[...]
  p1 = jnp.dot(pos2, w1p_r[...], preferred_element_type=_f32)
  z1_r[...] = jnp.dot(x, w1x_r[...], preferred_element_type=_f32) + p1
  p1_r[...] = p1
  xl = jnp.dot(x, wl_r[...], preferred_element_type=_f32) + bl_r[...]
  skip_r[...] = _bn(xl, gl_r[...], bel_r[...])


def _mid_body(m1_r, p1_r, b1_r, g1_r, be1_r, pos2_r, w2x_r, w2p_r,
              z2_r, p2_r):
  agg = m1_r[...]
  c1 = jnp.where(jnp.isneginf(agg), 0.0, agg - p1_r[...] + b1_r[...])
  h = jax.nn.relu(_bn(c1, g1_r[...], be1_r[...]))
  p2 = jnp.dot(pos2_r[...], w2p_r[...], preferred_element_type=_f32)
  z2_r[...] = jnp.dot(h, w2x_r[...], preferred_element_type=_f32) + p2
  p2_r[...] = p2


def _fin_body(m2_r, p2_r, b2_r, g2_r, be2_r, skip_r, out_r):
  agg = m2_r[...]
  c2 = jnp.where(jnp.isneginf(agg), 0.0, agg - p2_r[...] + b2_r[...])
  out_r[...] = jax.nn.relu(_bn(c2, g2_r[...], be2_r[...]) + skip_r[...])


def _tc_call(body, n_out):
  return pl.pallas_call(
      body,
      out_shape=tuple(jax.ShapeDtypeStruct((N, D), _f32)
                      for _ in range(n_out)),
  )


def kernel(x, pos, edge_index, W1, b1, g1, be1, W2, b2, g2, be2,
           Wl, bl, gl, bel):
  pos2 = pos[:, :2]
  src = edge_index[0].astype(_i32)
  dst = edge_index[1].astype(_i32)
  b1_, g1_, be1_ = b1.reshape(1, D), g1.reshape(1, D), be1.reshape(1, D)
  b2_, g2_, be2_ = b2.reshape(1, D), g2.reshape(1, D), be2.reshape(1, D)
  bl_, gl_, bel_ = bl.reshape(1, D), gl.reshape(1, D), bel.reshape(1, D)

  z1, p1, skip = _tc_call(_prep_body, 3)(
      x, pos2, W1[:D], W1[D:], Wl, bl_, gl_, bel_)

  m1 = _segmax(z1, src, dst).reshape(NPAD, D)[:N]

  z2, p2 = _tc_call(_mid_body, 2)(
      m1, p1, b1_, g1_, be1_, pos2, W2[:D], W2[D:])

  m2 = _segmax(z2, src, dst).reshape(NPAD, D)[:N]

  (out,) = _tc_call(_fin_body, 1)(m2, p2, b2_, g2_, be2_, skip)
  return out
